# baseline (speedup 1.0000x reference)
import functools

import jax
import jax.numpy as jnp
from jax import lax
from jax.experimental import pallas as pl
from jax.experimental.pallas import tpu as pltpu
from jax.experimental.pallas import tpu_sc as plsc

CAP = 1000000
B = 131072
NCOL = 46
NPAD = 128

NC = 2
NS = 16
RANGE_BITS = 15
RANGE = 1 << RANGE_BITS
ACTIVE_W = (CAP + RANGE - 1) // RANGE
OUT_ROWS = CAP + 2000

IDX_CHUNK = 16384
FLUSH_T = 112
FLUSH_CAP = 128

_WIDTHS = (3, 4, 3, 1, 3, 32)


def _copy_body(m_ref, q_ref, s_ref, o_ref, c_ref, l_ref, out_ref):
    tile = m_ref.shape[0]
    out_ref[...] = jnp.concatenate(
        [m_ref[...], q_ref[...], s_ref[...], o_ref[...], c_ref[...],
         l_ref[...], jnp.zeros((tile, NPAD - NCOL), jnp.float32)], axis=-1)


def _concat_copy(m, q, s, o, c, l, rows, tile):
    grid = rows // tile
    in_specs = [pl.BlockSpec((tile, w), lambda i: (i, i * 0))
                for w in _WIDTHS]
    out_rows = OUT_ROWS if rows == CAP else rows
    return pl.pallas_call(
        _copy_body,
        out_shape=jax.ShapeDtypeStruct((out_rows, NPAD), jnp.float32),
        grid=(grid,),
        in_specs=in_specs,
        out_specs=pl.BlockSpec((tile, NPAD), lambda i: (i, i * 0)),
    )(m, q, s, o, c, l)


def _fori(n, body):
    def wrapped(i, carry):
        body(i)
        return carry
    lax.fori_loop(jnp.int32(0), jnp.int32(n), wrapped, jnp.int32(0))


def _sc_scatter_body(idx_hbm, p_hbm, out_hbm,
                     w_ref, stage_ref, wini_ref, wind_ref, pay_ref,
                     sem, sem2):
    wid = lax.axis_index("s") * NC + lax.axis_index("c")

    @pl.when(wid < ACTIVE_W)
    def _():
        base = wid * RANGE
        lanes = lax.iota(jnp.int32, 16)
        widv = jnp.full((16,), wid, dtype=jnp.int32)

        dump = CAP + wid

        def init_w(i):
            w_ref[pl.ds(i * 16, 16)] = jnp.zeros((16,), jnp.int32)
        _fori(RANGE // 16, init_w)

        def init_f(i):
            wini_ref[pl.ds(i * 16, 16)] = jnp.zeros((16,), jnp.int32)
            wind_ref[pl.ds(i * 16, 16)] = jnp.full((16,), dump, jnp.int32)
        _fori(FLUSH_CAP // 16, init_f)

        nchunks = B // IDX_CHUNK

        for cj in range(nchunks):
            def do_chunk(chunk=cj):
                pltpu.sync_copy(idx_hbm.at[pl.ds(chunk * IDX_CHUNK,
                                                 IDX_CHUNK)], stage_ref)
                cbase = chunk * IDX_CHUNK + 1

                def group(g):
                    idxv = stage_ref[pl.ds(g * 16, 16)]
                    mask = (idxv >> RANGE_BITS) == widv
                    d = idxv & (RANGE - 1)
                    iv = cbase + g * 16 + lanes
                    cur = plsc.load_gather(w_ref, [d], mask=mask)
                    pending = mask & (iv > cur)

                    def cond(p):
                        return plsc.all_reduce_population_count(p)[0] > 0

                    def body(p):
                        plsc.store_scatter(w_ref, [d], iv, mask=p)
                        c2 = plsc.load_gather(w_ref, [d], mask=p)
                        return p & (iv > c2)

                    lax.while_loop(cond, body, pending)

                _fori(IDX_CHUNK // 16, group)
            do_chunk()

        def flush():
            pltpu.async_copy(p_hbm.at[wini_ref], pay_ref, sem).wait()
            pltpu.async_copy(pay_ref, out_hbm.at[wind_ref], sem2).wait()

        def group2(g, n):
            wv = w_ref[pl.ds(g * 16, 16)]
            m = wv > 0
            cnt = plsc.all_reduce_population_count(m)[0]
            plsc.store_compressed(wini_ref.at[pl.ds(n, 16)], wv - 1, mask=m)
            dest = base + g * 16 + lanes
            plsc.store_compressed(wind_ref.at[pl.ds(n, 16)], dest, mask=m)
            n = n + cnt

            def do_flush(nn):
                flush()
                return jnp.zeros_like(nn)

            return lax.cond(n >= FLUSH_T, do_flush, lambda nn: nn, n)

        lax.fori_loop(jnp.int32(0), jnp.int32(RANGE // 16), group2,
                      jnp.int32(0))
        flush()


@functools.cache
def _make_sc_scatter():
    return pl.kernel(
        _sc_scatter_body,
        out_type=(),
        mesh=plsc.VectorSubcoreMesh(core_axis_name="c",
                                    subcore_axis_name="s",
                                    num_cores=NC, num_subcores=NS),
        compiler_params=pltpu.CompilerParams(use_tc_tiling_on_sc=False,
                                             needs_layout_passes=False),
        scratch_types=[
            pltpu.VMEM((RANGE,), jnp.int32),
            pltpu.VMEM((IDX_CHUNK,), jnp.int32),
            pltpu.VMEM((FLUSH_CAP,), jnp.int32),
            pltpu.VMEM((FLUSH_CAP,), jnp.int32),
            pltpu.VMEM((FLUSH_CAP, NPAD), jnp.float32),
            pltpu.SemaphoreType.DMA,
            pltpu.SemaphoreType.DMA,
        ],
    )


def kernel(means3d_store, quat_store, log_scale_store, opacity_store,
           rgb_store, latent_store, new_means, new_quat, new_log_scale,
           new_opacity, new_rgb, new_latent, idx):
    idx32 = idx.astype(jnp.int32)
    out0 = _concat_copy(means3d_store, quat_store, log_scale_store,
                        opacity_store, rgb_store, latent_store,
                        rows=CAP, tile=2000)
    payload = _concat_copy(new_means, new_quat, new_log_scale, new_opacity,
                           new_rgb, new_latent, rows=B, tile=2048)
    out_ref = jax.new_ref(out0)
    _make_sc_scatter()(idx32, payload, out_ref)
    return jax.freeze(out_ref)[:CAP, :NCOL]


# speedup vs baseline: 2.1489x; 2.1489x over previous
import functools

import jax
import jax.numpy as jnp
from jax import lax
from jax.experimental import pallas as pl
from jax.experimental.pallas import tpu as pltpu
from jax.experimental.pallas import tpu_sc as plsc

CAP = 1000000
B = 131072
NCOL = 46
NPAD = 128

NC = 2
NS = 16
RANGE_BITS = 15
RANGE = 1 << RANGE_BITS
ACTIVE_W = (CAP + RANGE - 1) // RANGE
OUT_ROWS = CAP + 2000

IDX_CHUNK = 16384
FLUSH_T = 112
FLUSH_CAP = 128

_WIDTHS = (3, 4, 3, 1, 3, 32)


def _copy_body(m_ref, q_ref, s_ref, o_ref, c_ref, l_ref, out_ref):
    tile = m_ref.shape[1]
    cat = jnp.concatenate(
        [m_ref[...], q_ref[...], s_ref[...], o_ref[...], c_ref[...],
         l_ref[...], jnp.zeros((NPAD - NCOL, tile), jnp.float32)], axis=0)
    out_ref[...] = cat.T


def _concat_copy(m, q, s, o, c, l, rows, tile):
    grid = -(-rows // tile)
    in_specs = [pl.BlockSpec((w, tile), lambda i: (i * 0, i))
                for w in _WIDTHS]
    out_rows = OUT_ROWS if rows == CAP else rows
    return pl.pallas_call(
        _copy_body,
        out_shape=jax.ShapeDtypeStruct((out_rows, NPAD), jnp.float32),
        grid=(grid,),
        in_specs=in_specs,
        out_specs=pl.BlockSpec((tile, NPAD), lambda i: (i, i * 0)),
    )(m.T, q.T, s.T, o.T, c.T, l.T)


def _fori(n, body):
    def wrapped(i, carry):
        body(i)
        return carry
    lax.fori_loop(jnp.int32(0), jnp.int32(n), wrapped, jnp.int32(0))


def _sc_scatter_body(idx_hbm, p_hbm, out_hbm,
                     w_ref, stage_ref, wini_ref, wind_ref, pay_ref,
                     sem, sem2):
    wid = lax.axis_index("s") * NC + lax.axis_index("c")

    @pl.when(wid < ACTIVE_W)
    def _():
        base = wid * RANGE
        lanes = lax.iota(jnp.int32, 16)
        widv = jnp.full((16,), wid, dtype=jnp.int32)

        dump = CAP + wid

        def init_w(i):
            w_ref[pl.ds(i * 16, 16)] = jnp.zeros((16,), jnp.int32)
        _fori(RANGE // 16, init_w)

        def init_f(i):
            wini_ref[pl.ds(i * 16, 16)] = jnp.zeros((16,), jnp.int32)
            wind_ref[pl.ds(i * 16, 16)] = jnp.full((16,), dump, jnp.int32)
        _fori(FLUSH_CAP // 16, init_f)

        nchunks = B // IDX_CHUNK

        for cj in range(nchunks):
            def do_chunk(chunk=cj):
                pltpu.sync_copy(idx_hbm.at[pl.ds(chunk * IDX_CHUNK,
                                                 IDX_CHUNK)], stage_ref)
                cbase = chunk * IDX_CHUNK + 1

                def group(g):
                    idxv = stage_ref[pl.ds(g * 16, 16)]
                    mask = (idxv >> RANGE_BITS) == widv
                    d = idxv & (RANGE - 1)
                    iv = cbase + g * 16 + lanes
                    cur = plsc.load_gather(w_ref, [d], mask=mask)
                    pending = mask & (iv > cur)

                    def cond(p):
                        return plsc.all_reduce_population_count(p)[0] > 0

                    def body(p):
                        plsc.store_scatter(w_ref, [d], iv, mask=p)
                        c2 = plsc.load_gather(w_ref, [d], mask=p)
                        return p & (iv > c2)

                    lax.while_loop(cond, body, pending)

                _fori(IDX_CHUNK // 16, group)
            do_chunk()

        def flush():
            pltpu.async_copy(p_hbm.at[wini_ref], pay_ref, sem).wait()
            pltpu.async_copy(pay_ref, out_hbm.at[wind_ref], sem2).wait()

        def group2(g, n):
            wv = w_ref[pl.ds(g * 16, 16)]
            m = wv > 0
            cnt = plsc.all_reduce_population_count(m)[0]
            plsc.store_compressed(wini_ref.at[pl.ds(n, 16)], wv - 1, mask=m)
            dest = base + g * 16 + lanes
            plsc.store_compressed(wind_ref.at[pl.ds(n, 16)], dest, mask=m)
            n = n + cnt

            def do_flush(nn):
                flush()
                return jnp.zeros_like(nn)

            return lax.cond(n >= FLUSH_T, do_flush, lambda nn: nn, n)

        lax.fori_loop(jnp.int32(0), jnp.int32(RANGE // 16), group2,
                      jnp.int32(0))
        flush()


@functools.cache
def _make_sc_scatter():
    return pl.kernel(
        _sc_scatter_body,
        out_type=(),
        mesh=plsc.VectorSubcoreMesh(core_axis_name="c",
                                    subcore_axis_name="s",
                                    num_cores=NC, num_subcores=NS),
        compiler_params=pltpu.CompilerParams(use_tc_tiling_on_sc=False,
                                             needs_layout_passes=False),
        scratch_types=[
            pltpu.VMEM((RANGE,), jnp.int32),
            pltpu.VMEM((IDX_CHUNK,), jnp.int32),
            pltpu.VMEM((FLUSH_CAP,), jnp.int32),
            pltpu.VMEM((FLUSH_CAP,), jnp.int32),
            pltpu.VMEM((FLUSH_CAP, NPAD), jnp.float32),
            pltpu.SemaphoreType.DMA,
            pltpu.SemaphoreType.DMA,
        ],
    )


def kernel(means3d_store, quat_store, log_scale_store, opacity_store,
           rgb_store, latent_store, new_means, new_quat, new_log_scale,
           new_opacity, new_rgb, new_latent, idx):
    idx32 = idx.astype(jnp.int32)
    out0 = _concat_copy(means3d_store, quat_store, log_scale_store,
                        opacity_store, rgb_store, latent_store,
                        rows=CAP, tile=2048)
    payload = _concat_copy(new_means, new_quat, new_log_scale, new_opacity,
                           new_rgb, new_latent, rows=B, tile=2048)
    out_ref = jax.new_ref(out0)
    _make_sc_scatter()(idx32, payload, out_ref)
    return jax.freeze(out_ref)[:CAP, :NCOL]


# speedup vs baseline: 2.1744x; 1.0119x over previous
import functools

import jax
import jax.numpy as jnp
from jax import lax
from jax.experimental import pallas as pl
from jax.experimental.pallas import tpu as pltpu
from jax.experimental.pallas import tpu_sc as plsc

CAP = 1000000
B = 131072
NCOL = 46
NPAD = 128

NC = 2
NS = 16
RANGE_BITS = 15
RANGE = 1 << RANGE_BITS
ACTIVE_W = (CAP + RANGE - 1) // RANGE
OUT_ROWS = CAP + 2000

IDX_CHUNK = 16384
FLUSH_T = 112
FLUSH_CAP = 128

_WIDTHS = (3, 4, 3, 1, 3, 32)


def _copy_body(m_ref, q_ref, s_ref, o_ref, c_ref, l_ref, out_ref):
    tile = m_ref.shape[1]
    cat = jnp.concatenate(
        [m_ref[...], q_ref[...], s_ref[...], o_ref[...], c_ref[...],
         l_ref[...], jnp.zeros((NPAD - NCOL, tile), jnp.float32)], axis=0)
    out_ref[...] = cat.T


def _concat_copy(m, q, s, o, c, l, rows, tile):
    grid = -(-rows // tile)
    in_specs = [pl.BlockSpec((w, tile), lambda i: (i * 0, i))
                for w in _WIDTHS]
    out_rows = OUT_ROWS if rows == CAP else rows
    return pl.pallas_call(
        _copy_body,
        out_shape=jax.ShapeDtypeStruct((out_rows, NPAD), jnp.float32),
        grid=(grid,),
        in_specs=in_specs,
        out_specs=pl.BlockSpec((tile, NPAD), lambda i: (i, i * 0)),
    )(m.T, q.T, s.T, o.T, c.T, l.T)


def _fori(n, body):
    def wrapped(i, carry):
        body(i)
        return carry
    lax.fori_loop(jnp.int32(0), jnp.int32(n), wrapped, jnp.int32(0))


def _sc_scatter_body(idx_hbm, p_hbm, out_hbm,
                     w_ref, stage_ref, wini_ref, wind_ref, pay_ref,
                     sem, sem2):
    wid = lax.axis_index("s") * NC + lax.axis_index("c")

    @pl.when(wid < ACTIVE_W)
    def _():
        base = wid * RANGE
        lanes = lax.iota(jnp.int32, 16)
        widv = jnp.full((16,), wid, dtype=jnp.int32)

        dump = CAP + wid

        def init_w(i):
            w_ref[pl.ds(i * 16, 16)] = jnp.zeros((16,), jnp.int32)
        _fori(RANGE // 16, init_w)

        def init_f(i):
            wini_ref[pl.ds(i * 16, 16)] = jnp.zeros((16,), jnp.int32)
            wind_ref[pl.ds(i * 16, 16)] = jnp.full((16,), dump, jnp.int32)
        _fori(FLUSH_CAP // 16, init_f)

        nchunks = B // IDX_CHUNK

        for cj in range(nchunks):
            def do_chunk(chunk=cj):
                pltpu.sync_copy(idx_hbm.at[pl.ds(chunk * IDX_CHUNK,
                                                 IDX_CHUNK)], stage_ref)
                cbase = chunk * IDX_CHUNK + 1

                def scan_pass(_):
                    def group(g, sticky):
                        idxv = stage_ref[pl.ds(g * 16, 16)]
                        mask = (idxv >> RANGE_BITS) == widv
                        d = idxv & (RANGE - 1)
                        iv = cbase + g * 16 + lanes
                        cur = plsc.load_gather(w_ref, [d], mask=mask)
                        pending = mask & (iv > cur)
                        plsc.store_scatter(w_ref, [d], iv, mask=pending)
                        c2 = plsc.load_gather(w_ref, [d], mask=pending)
                        return sticky | (pending & (iv > c2))

                    sticky = lax.fori_loop(
                        jnp.int32(0), jnp.int32(IDX_CHUNK // 16), group,
                        jnp.zeros((16,), bool))
                    return plsc.all_reduce_population_count(sticky)[0]

                lax.while_loop(lambda c: c > 0, scan_pass, jnp.int32(1))
            do_chunk()

        def flush():
            pltpu.async_copy(p_hbm.at[wini_ref], pay_ref, sem).wait()
            pltpu.async_copy(pay_ref, out_hbm.at[wind_ref], sem2).wait()

        def group2(g, n):
            wv = w_ref[pl.ds(g * 16, 16)]
            m = wv > 0
            cnt = plsc.all_reduce_population_count(m)[0]
            plsc.store_compressed(wini_ref.at[pl.ds(n, 16)], wv - 1, mask=m)
            dest = base + g * 16 + lanes
            plsc.store_compressed(wind_ref.at[pl.ds(n, 16)], dest, mask=m)
            n = n + cnt

            def do_flush(nn):
                flush()
                return jnp.zeros_like(nn)

            return lax.cond(n >= FLUSH_T, do_flush, lambda nn: nn, n)

        lax.fori_loop(jnp.int32(0), jnp.int32(RANGE // 16), group2,
                      jnp.int32(0))
        flush()


@functools.cache
def _make_sc_scatter():
    return pl.kernel(
        _sc_scatter_body,
        out_type=(),
        mesh=plsc.VectorSubcoreMesh(core_axis_name="c",
                                    subcore_axis_name="s",
                                    num_cores=NC, num_subcores=NS),
        compiler_params=pltpu.CompilerParams(use_tc_tiling_on_sc=False,
                                             needs_layout_passes=False),
        scratch_types=[
            pltpu.VMEM((RANGE,), jnp.int32),
            pltpu.VMEM((IDX_CHUNK,), jnp.int32),
            pltpu.VMEM((FLUSH_CAP,), jnp.int32),
            pltpu.VMEM((FLUSH_CAP,), jnp.int32),
            pltpu.VMEM((FLUSH_CAP, NPAD), jnp.float32),
            pltpu.SemaphoreType.DMA,
            pltpu.SemaphoreType.DMA,
        ],
    )


def kernel(means3d_store, quat_store, log_scale_store, opacity_store,
           rgb_store, latent_store, new_means, new_quat, new_log_scale,
           new_opacity, new_rgb, new_latent, idx):
    idx32 = idx.astype(jnp.int32)
    out0 = _concat_copy(means3d_store, quat_store, log_scale_store,
                        opacity_store, rgb_store, latent_store,
                        rows=CAP, tile=2048)
    payload = _concat_copy(new_means, new_quat, new_log_scale, new_opacity,
                           new_rgb, new_latent, rows=B, tile=2048)
    out_ref = jax.new_ref(out0)
    _make_sc_scatter()(idx32, payload, out_ref)
    return jax.freeze(out_ref)[:CAP, :NCOL]


# speedup vs baseline: 2.4271x; 1.1162x over previous
import functools

import jax
import jax.numpy as jnp
from jax import lax
from jax.experimental import pallas as pl
from jax.experimental.pallas import tpu as pltpu
from jax.experimental.pallas import tpu_sc as plsc

CAP = 1000000
B = 131072
NCOL = 46
NPAD = 128

NC = 2
NS = 16
RANGE_BITS = 15
RANGE = 1 << RANGE_BITS
ACTIVE_W = (CAP + RANGE - 1) // RANGE
OUT_ROWS = CAP + 2000

IDX_CHUNK = 16384
FLUSH_T = 112
FLUSH_CAP = 128

_WIDTHS = (3, 4, 3, 1, 3, 32)


def _copy_body(m_ref, q_ref, s_ref, o_ref, c_ref, l_ref, out_ref):
    tile = m_ref.shape[1]
    cat = jnp.concatenate(
        [m_ref[...], q_ref[...], s_ref[...], o_ref[...], c_ref[...],
         l_ref[...], jnp.zeros((NPAD - NCOL, tile), jnp.float32)], axis=0)
    out_ref[...] = cat.T


def _concat_copy(m, q, s, o, c, l, rows, tile):
    grid = -(-rows // tile)
    in_specs = [pl.BlockSpec((w, tile), lambda i: (i * 0, i))
                for w in _WIDTHS]
    out_rows = OUT_ROWS if rows == CAP else rows
    return pl.pallas_call(
        _copy_body,
        out_shape=jax.ShapeDtypeStruct((out_rows, NPAD), jnp.float32),
        grid=(grid,),
        in_specs=in_specs,
        out_specs=pl.BlockSpec((tile, NPAD), lambda i: (i, i * 0)),
    )(m.T, q.T, s.T, o.T, c.T, l.T)


def _fori(n, body):
    def wrapped(i, carry):
        body(i)
        return carry
    lax.fori_loop(jnp.int32(0), jnp.int32(n), wrapped, jnp.int32(0))


def _sc_scatter_body(idx_hbm, p_hbm, out_hbm,
                     w_ref, stage_ref, wini_ref, wind_ref, pay_ref,
                     sem, sem2):
    wid = lax.axis_index("s") * NC + lax.axis_index("c")

    @pl.when(wid < ACTIVE_W)
    def _():
        base = wid * RANGE
        lanes = lax.iota(jnp.int32, 16)
        widv = jnp.full((16,), wid, dtype=jnp.int32)

        dump = CAP + wid

        def init_w(i):
            w_ref[pl.ds(i * 16, 16)] = jnp.zeros((16,), jnp.int32)
        _fori(RANGE // 16, init_w)

        def init_f(i):
            wini_ref[pl.ds(i * 16, 16)] = jnp.zeros((16,), jnp.int32)
            wind_ref[pl.ds(i * 16, 16)] = jnp.full((16,), dump, jnp.int32)
        _fori(FLUSH_CAP // 16, init_f)

        nchunks = B // IDX_CHUNK

        for cj in range(nchunks):
            def do_chunk(chunk=cj):
                pltpu.sync_copy(idx_hbm.at[pl.ds(chunk * IDX_CHUNK,
                                                 IDX_CHUNK)], stage_ref)
                cbase = chunk * IDX_CHUNK + 1

                def scan_pass(_):
                    def group(g, sticky):
                        idxv = stage_ref[pl.ds(g * 16, 16)]
                        mask = (idxv >> RANGE_BITS) == widv
                        d = idxv & (RANGE - 1)
                        iv = cbase + g * 16 + lanes
                        cur = plsc.load_gather(w_ref, [d], mask=mask)
                        pending = mask & (iv > cur)
                        plsc.store_scatter(w_ref, [d], iv, mask=pending)
                        c2 = plsc.load_gather(w_ref, [d], mask=pending)
                        return sticky | (pending & (iv > c2))

                    sticky = lax.fori_loop(
                        jnp.int32(0), jnp.int32(IDX_CHUNK // 16), group,
                        jnp.zeros((16,), bool))
                    return plsc.all_reduce_population_count(sticky)[0]

                lax.while_loop(lambda c: c > 0, scan_pass, jnp.int32(1))
            do_chunk()

        def flush():
            pltpu.async_copy(p_hbm.at[wini_ref], pay_ref, sem).wait()
            pltpu.async_copy(pay_ref, out_hbm.at[wind_ref], sem2).wait()

        def group2(g, n):
            wv = w_ref[pl.ds(g * 16, 16)]
            m = wv > 0
            cnt = plsc.all_reduce_population_count(m)[0]
            plsc.store_compressed(wini_ref.at[pl.ds(n, 16)], wv - 1, mask=m)
            dest = base + g * 16 + lanes
            plsc.store_compressed(wind_ref.at[pl.ds(n, 16)], dest, mask=m)
            n = n + cnt

            def do_flush(nn):
                flush()
                return jnp.zeros_like(nn)

            return lax.cond(n >= FLUSH_T, do_flush, lambda nn: nn, n)

        lax.fori_loop(jnp.int32(0), jnp.int32(RANGE // 16), group2,
                      jnp.int32(0))
        flush()


@functools.cache
def _make_sc_scatter():
    return pl.kernel(
        _sc_scatter_body,
        out_type=(),
        mesh=plsc.VectorSubcoreMesh(core_axis_name="c",
                                    subcore_axis_name="s",
                                    num_cores=NC, num_subcores=NS),
        compiler_params=pltpu.CompilerParams(use_tc_tiling_on_sc=False,
                                             needs_layout_passes=False),
        scratch_types=[
            pltpu.VMEM((RANGE,), jnp.int32),
            pltpu.VMEM((IDX_CHUNK,), jnp.int32),
            pltpu.VMEM((FLUSH_CAP,), jnp.int32),
            pltpu.VMEM((FLUSH_CAP,), jnp.int32),
            pltpu.VMEM((FLUSH_CAP, NPAD), jnp.float32),
            pltpu.SemaphoreType.DMA,
            pltpu.SemaphoreType.DMA,
        ],
    )


def kernel(means3d_store, quat_store, log_scale_store, opacity_store,
           rgb_store, latent_store, new_means, new_quat, new_log_scale,
           new_opacity, new_rgb, new_latent, idx):
    idx32 = idx.astype(jnp.int32)
    out0 = _concat_copy(means3d_store, quat_store, log_scale_store,
                        opacity_store, rgb_store, latent_store,
                        rows=CAP, tile=8192)
    payload = _concat_copy(new_means, new_quat, new_log_scale, new_opacity,
                           new_rgb, new_latent, rows=B, tile=8192)
    out_ref = jax.new_ref(out0)
    _make_sc_scatter()(idx32, payload, out_ref)
    return jax.freeze(out_ref)[:CAP, :NCOL]
